# Initial kernel scaffold; baseline (speedup 1.0000x reference)
import jax
import jax.numpy as jnp
from jax.experimental import pallas as pl


def kernel(x, edge_index, edge_weight, W, b, gamma, beta):
    raise NotImplementedError("write your pallas kernel here")


# speedup vs baseline: 10.7580x; 10.7580x over previous
import functools

import jax
import jax.numpy as jnp
from jax import lax
from jax.experimental import pallas as pl
from jax.experimental.pallas import tpu as pltpu
from jax.experimental.pallas import tpu_sc as plsc

NC = 2
NS = 16
NW = NC * NS
LANES = 16


def _zero_f32(ref, n):
  def body(i, carry):
    ref[pl.ds(i * LANES, LANES)] = jnp.zeros((LANES,), jnp.float32)
    return carry
  lax.fori_loop(0, n // LANES, body, 0)


def _make_deg_kernel(e_pad, np_):
  ch = 2048
  per_w = e_pad // NW
  n_chunks = per_w // ch
  slice_pw = np_ // NS
  mesh = plsc.VectorSubcoreMesh(core_axis_name="c", subcore_axis_name="s")

  @functools.partial(
      pl.kernel,
      out_type=jax.ShapeDtypeStruct((NC, 1, np_), jnp.float32),
      mesh=mesh,
      scratch_types=[
          pltpu.VMEM((ch // 128, 1, 128), jnp.int32),
          pltpu.VMEM((ch,), jnp.float32),
          pltpu.VMEM((slice_pw,), jnp.float32),
          pltpu.VMEM_SHARED((np_,), jnp.float32),
          pltpu.SemaphoreType.DMA,
      ],
  )
  def deg_kernel(dst3_hbm, w_hbm, out_hbm, dstv, wv, zbuf, acc, sem):
    del sem
    c = lax.axis_index("c")
    s = lax.axis_index("s")
    wid = s * NC + c
    _zero_f32(zbuf, slice_pw)
    pltpu.sync_copy(zbuf, acc.at[pl.ds(s * slice_pw, slice_pw)])
    plsc.subcore_barrier()

    def chunk(k, carry):
      base = wid * per_w + k * ch
      pltpu.sync_copy(w_hbm.at[pl.ds(base, ch)], wv)
      pltpu.sync_copy(dst3_hbm.at[pl.ds(base // 128, ch // 128)], dstv)

      def sub(j, carry2):
        pltpu.sync_copy(wv.at[pl.ds(j * 128, 128)], acc.at[dstv.at[j, 0]],
                        add=True)
        return carry2
      lax.fori_loop(0, ch // 128, sub, 0)
      return carry
    lax.fori_loop(0, n_chunks, chunk, 0)

    plsc.subcore_barrier()
    pltpu.sync_copy(acc.at[pl.ds(s * slice_pw, slice_pw)],
                    out_hbm.at[c, 0, pl.ds(s * slice_pw, slice_pw)])

  return deg_kernel


def _make_msg_kernel(e_pad, np_, d):
  blk = 1024
  half = 256
  per_w = e_pad // NW
  n_blocks = per_w // blk
  rows_pw = np_ // NS
  mesh = plsc.VectorSubcoreMesh(core_axis_name="c", subcore_axis_name="s")

  @functools.partial(
      pl.kernel,
      out_type=jax.ShapeDtypeStruct((NC, np_, d), jnp.float32),
      mesh=mesh,
      scratch_types=[
          pltpu.VMEM((blk,), jnp.int32),
          pltpu.VMEM((blk // 128, 1, 128), jnp.int32),
          pltpu.VMEM((blk,), jnp.float32),
          pltpu.VMEM((half, d), jnp.float32),
          pltpu.VMEM_SHARED((np_, d), jnp.float32),
          pltpu.SemaphoreType.DMA,
      ],
  )
  def msg_kernel(src_hbm, dst3_hbm, w_hbm, xw2_hbm, out_hbm,
                 srcv, dstv, wv, rows, acc, sem):
    c = lax.axis_index("c")
    s = lax.axis_index("s")
    wid = s * NC + c

    def zrow(i, carry):
      e = i // (d // LANES)
      j = i % (d // LANES)
      rows[e, pl.ds(j * LANES, LANES)] = jnp.zeros((LANES,), jnp.float32)
      return carry
    lax.fori_loop(0, half * d // LANES, zrow, 0)
    off = 0
    while off < rows_pw:
      sz = min(half, rows_pw - off)
      pltpu.sync_copy(rows.at[pl.ds(0, sz)],
                      acc.at[pl.ds(s * rows_pw + off, sz)])
      off += sz
    plsc.subcore_barrier()

    def block(k, carry):
      ebase = wid * per_w + k * blk
      pltpu.sync_copy(src_hbm.at[pl.ds(ebase, blk)], srcv)
      pltpu.sync_copy(w_hbm.at[pl.ds(ebase, blk)], wv)
      pltpu.sync_copy(dst3_hbm.at[pl.ds(ebase // 128, blk // 128)], dstv)

      for h in range(blk // half):
        pltpu.async_copy(xw2_hbm.at[srcv.at[pl.ds(h * half, half)]],
                         rows, sem).wait()

        def scale(g, carry2):
          wvec = wv[pl.ds(h * half + g * LANES, LANES)]
          e0 = g * LANES
          for l in range(LANES):
            we = wvec[l]
            for j in range(d // LANES):
              rows[e0 + l, pl.ds(j * LANES, LANES)] = (
                  rows[e0 + l, pl.ds(j * LANES, LANES)] * we)
          return carry2
        lax.fori_loop(0, half // LANES, scale, 0)

        def sub(j, carry2):
          pltpu.sync_copy(rows.at[pl.ds(j * 128, 128)],
                          acc.at[dstv.at[h * (half // 128) + j, 0]], add=True)
          return carry2
        lax.fori_loop(0, half // 128, sub, 0)
      return carry
    lax.fori_loop(0, n_blocks, block, 0)

    plsc.subcore_barrier()
    pltpu.sync_copy(acc.at[pl.ds(s * rows_pw, rows_pw)],
                    out_hbm.at[c, pl.ds(s * rows_pw, rows_pw)])

  return msg_kernel


def _mm_body(x_ref, w_ref, d0_ref, d1_ref, xw2_ref, dis_ref):
  deg = d0_ref[...] + d1_ref[...] + 1.0
  dis = lax.rsqrt(deg)
  xw = jnp.dot(x_ref[...], w_ref[...],
               preferred_element_type=jnp.float32,
               precision=lax.Precision.HIGHEST)
  xw2_ref[...] = xw * dis
  dis_ref[...] = dis


def _bn_body(a0_ref, a1_ref, xw2_ref, dis_ref, b_ref, g_ref, be_ref, out_ref):
  n = a0_ref.shape[0]
  t = (a0_ref[...] + a1_ref[...] + xw2_ref[...]) * dis_ref[...] + b_ref[...]
  mean = jnp.sum(t, axis=0, keepdims=True) * (1.0 / n)
  tc = t - mean
  var = jnp.sum(tc * tc, axis=0, keepdims=True) * (1.0 / n)
  h = tc * lax.rsqrt(var + 1e-5) * g_ref[...] + be_ref[...]
  out_ref[...] = jnp.maximum(h, 0.0)


def kernel(x, edge_index, edge_weight, W, b, gamma, beta):
  n, d_in = x.shape
  d_out = W.shape[1]
  e = edge_weight.shape[0]

  unit = NW * 2048
  e_pad = -(-e // unit) * unit
  np_ = -(-n // (NS * 640)) * (NS * 640)

  pad = e_pad - e
  src_p = jnp.concatenate([edge_index[0], jnp.zeros((pad,), jnp.int32)])
  dst_p = jnp.concatenate([edge_index[1], jnp.zeros((pad,), jnp.int32)])
  w_p = jnp.concatenate([edge_weight, jnp.zeros((pad,), jnp.float32)])
  dst3 = dst_p.reshape(e_pad // 128, 1, 128)

  degp = _make_deg_kernel(e_pad, np_)(dst3, w_p)

  dp0 = degp[0, 0, :n].reshape(n, 1)
  dp1 = degp[1, 0, :n].reshape(n, 1)
  xw2, dis = pl.pallas_call(
      _mm_body,
      out_shape=[
          jax.ShapeDtypeStruct((n, d_out), jnp.float32),
          jax.ShapeDtypeStruct((n, 1), jnp.float32),
      ],
  )(x, W, dp0, dp1)

  accp = _make_msg_kernel(e_pad, np_, d_out)(src_p, dst3, w_p, xw2)

  out = pl.pallas_call(
      _bn_body,
      out_shape=jax.ShapeDtypeStruct((n, d_out), jnp.float32),
  )(accp[0, :n], accp[1, :n], xw2, dis,
    b.reshape(1, d_out), gamma.reshape(1, d_out), beta.reshape(1, d_out))
  return out


# speedup vs baseline: 12.2021x; 1.1342x over previous
import functools

import jax
import jax.numpy as jnp
from jax import lax
from jax.experimental import pallas as pl
from jax.experimental.pallas import tpu as pltpu
from jax.experimental.pallas import tpu_sc as plsc

NC = 2
NS = 16
NW = NC * NS
LANES = 16


def _zero_f32(ref, n):
  def body(i, carry):
    ref[pl.ds(i * LANES, LANES)] = jnp.zeros((LANES,), jnp.float32)
    return carry
  lax.fori_loop(0, n // LANES, body, 0)


def _make_deg_kernel(e_pad, np_):
  ch = 2048
  per_w = e_pad // NW
  n_chunks = per_w // ch
  slice_pw = np_ // NS
  mesh = plsc.VectorSubcoreMesh(core_axis_name="c", subcore_axis_name="s")

  @functools.partial(
      pl.kernel,
      out_type=jax.ShapeDtypeStruct((NC, 1, np_), jnp.float32),
      mesh=mesh,
      scratch_types=[
          pltpu.VMEM((ch // 128, 1, 128), jnp.int32),
          pltpu.VMEM((ch,), jnp.float32),
          pltpu.VMEM((slice_pw,), jnp.float32),
          pltpu.VMEM_SHARED((np_,), jnp.float32),
          pltpu.SemaphoreType.DMA,
      ],
  )
  def deg_kernel(dst3_hbm, w_hbm, out_hbm, dstv, wv, zbuf, acc, sem):
    del sem
    c = lax.axis_index("c")
    s = lax.axis_index("s")
    wid = s * NC + c
    _zero_f32(zbuf, slice_pw)
    pltpu.sync_copy(zbuf, acc.at[pl.ds(s * slice_pw, slice_pw)])
    plsc.subcore_barrier()

    def chunk(k, carry):
      base = wid * per_w + k * ch
      pltpu.sync_copy(w_hbm.at[pl.ds(base, ch)], wv)
      pltpu.sync_copy(dst3_hbm.at[pl.ds(base // 128, ch // 128)], dstv)

      def sub(j, carry2):
        pltpu.sync_copy(wv.at[pl.ds(j * 128, 128)], acc.at[dstv.at[j, 0]],
                        add=True)
        return carry2
      lax.fori_loop(0, ch // 128, sub, 0)
      return carry
    lax.fori_loop(0, n_chunks, chunk, 0)

    plsc.subcore_barrier()
    pltpu.sync_copy(acc.at[pl.ds(s * slice_pw, slice_pw)],
                    out_hbm.at[c, 0, pl.ds(s * slice_pw, slice_pw)])

  return deg_kernel


def _make_msg_kernel(e_pad, np_, d):
  blk = 1024
  p = 64
  npc = blk // p
  nbuf = 4
  per_w = e_pad // NW
  n_blocks = per_w // blk
  rows_pw = np_ // NS
  mesh = plsc.VectorSubcoreMesh(core_axis_name="c", subcore_axis_name="s")

  @functools.partial(
      pl.kernel,
      out_type=jax.ShapeDtypeStruct((NC, np_, d), jnp.float32),
      mesh=mesh,
      scratch_types=[
          pltpu.VMEM((2, blk), jnp.int32),
          pltpu.VMEM((2, npc, 1, p), jnp.int32),
          pltpu.VMEM((2, blk), jnp.float32),
          [pltpu.VMEM((p, d), jnp.float32) for _ in range(nbuf)],
          pltpu.VMEM_SHARED((np_, d), jnp.float32),
          [pltpu.SemaphoreType.DMA for _ in range(nbuf)],
          [pltpu.SemaphoreType.DMA for _ in range(nbuf)],
          pltpu.SemaphoreType.DMA,
      ],
  )
  def msg_kernel(src_hbm, dst4_hbm, w_hbm, xw2_hbm, out_hbm,
                 srcv, dstv, wv, gbufs, acc, gsems, ssems, isem):
    c = lax.axis_index("c")
    s = lax.axis_index("s")
    wid = s * NC + c

    def zrow(i, carry):
      e = i // (d // LANES)
      j = i % (d // LANES)
      gbufs[0][e, pl.ds(j * LANES, LANES)] = jnp.zeros((LANES,), jnp.float32)
      return carry
    lax.fori_loop(0, p * d // LANES, zrow, 0)
    for off in range(0, rows_pw, p):
      pltpu.sync_copy(gbufs[0], acc.at[pl.ds(s * rows_pw + off, p)])
    plsc.subcore_barrier()

    def idx_copies(k, parity):
      ebase = wid * per_w + k * blk
      return [
          pltpu.make_async_copy(src_hbm.at[pl.ds(ebase, blk)],
                                srcv.at[parity], isem),
          pltpu.make_async_copy(w_hbm.at[pl.ds(ebase, blk)],
                                wv.at[parity], isem),
          pltpu.make_async_copy(dst4_hbm.at[pl.ds(ebase // p, npc)],
                                dstv.at[parity], isem),
      ]

    for cp in idx_copies(0, 0):
      cp.start()

    def block(k, carry):
      parity = lax.rem(k, 2)
      for cp in idx_copies(k, parity):
        cp.wait()

      @pl.when(k + 1 < n_blocks)
      def _():
        for cp in idx_copies(k + 1, 1 - parity):
          cp.start()

      def gath(i, b):
        return pltpu.make_async_copy(
            xw2_hbm.at[srcv.at[parity, pl.ds(i * p, p)]], gbufs[b], gsems[b])

      def scat(i, b):
        return pltpu.make_async_copy(gbufs[b], acc.at[dstv.at[parity, i, 0]],
                                     ssems[b])

      gath(0, 0).start()
      gath(1, 1).start()
      sdescs = [None] * npc
      for i in range(npc):
        b = i % nbuf
        gath(i, b).wait()

        def scale(g, carry2):
          wvec = wv[parity, pl.ds(i * p + g * LANES, LANES)]
          e0 = g * LANES
          for l in range(LANES):
            we = wvec[l]
            for j in range(d // LANES):
              gbufs[b][e0 + l, pl.ds(j * LANES, LANES)] = (
                  gbufs[b][e0 + l, pl.ds(j * LANES, LANES)] * we)
          return carry2
        lax.fori_loop(0, p // LANES, scale, 0)

        sd = scat(i, b)
        sd.start(add=True)
        sdescs[i] = sd
        if i + 2 < npc:
          b2 = (i + 2) % nbuf
          if i - 2 >= 0:
            sdescs[i - 2].wait()
          gath(i + 2, b2).start()
      for i in range(npc - 4, npc):
        sdescs[i].wait()
      return carry
    lax.fori_loop(0, n_blocks, block, 0)

    plsc.subcore_barrier()
    pltpu.sync_copy(acc.at[pl.ds(s * rows_pw, rows_pw)],
                    out_hbm.at[c, pl.ds(s * rows_pw, rows_pw)])

  return msg_kernel


def _mm_body(x_ref, w_ref, d0_ref, d1_ref, xw2_ref, dis_ref):
  deg = d0_ref[...] + d1_ref[...] + 1.0
  dis = lax.rsqrt(deg)
  xw = jnp.dot(x_ref[...], w_ref[...],
               preferred_element_type=jnp.float32,
               precision=lax.Precision.HIGHEST)
  xw2_ref[...] = xw * dis
  dis_ref[...] = dis


def _bn_body(a0_ref, a1_ref, xw2_ref, dis_ref, b_ref, g_ref, be_ref, out_ref):
  n = a0_ref.shape[0]
  t = (a0_ref[...] + a1_ref[...] + xw2_ref[...]) * dis_ref[...] + b_ref[...]
  mean = jnp.sum(t, axis=0, keepdims=True) * (1.0 / n)
  tc = t - mean
  var = jnp.sum(tc * tc, axis=0, keepdims=True) * (1.0 / n)
  h = tc * lax.rsqrt(var + 1e-5) * g_ref[...] + be_ref[...]
  out_ref[...] = jnp.maximum(h, 0.0)


def kernel(x, edge_index, edge_weight, W, b, gamma, beta):
  n, d_in = x.shape
  d_out = W.shape[1]
  e = edge_weight.shape[0]

  unit = NW * 2048
  e_pad = -(-e // unit) * unit
  np_ = -(-n // (NS * 640)) * (NS * 640)

  pad = e_pad - e
  src_p = jnp.concatenate([edge_index[0], jnp.zeros((pad,), jnp.int32)])
  dst_p = jnp.concatenate([edge_index[1], jnp.zeros((pad,), jnp.int32)])
  w_p = jnp.concatenate([edge_weight, jnp.zeros((pad,), jnp.float32)])
  dst3 = dst_p.reshape(e_pad // 128, 1, 128)
  dst4 = dst_p.reshape(e_pad // 64, 1, 64)

  degp = _make_deg_kernel(e_pad, np_)(dst3, w_p)

  dp0 = degp[0, 0, :n].reshape(n, 1)
  dp1 = degp[1, 0, :n].reshape(n, 1)
  xw2, dis = pl.pallas_call(
      _mm_body,
      out_shape=[
          jax.ShapeDtypeStruct((n, d_out), jnp.float32),
          jax.ShapeDtypeStruct((n, 1), jnp.float32),
      ],
  )(x, W, dp0, dp1)

  accp = _make_msg_kernel(e_pad, np_, d_out)(src_p, dst4, w_p, xw2)

  out = pl.pallas_call(
      _bn_body,
      out_shape=jax.ShapeDtypeStruct((n, d_out), jnp.float32),
  )(accp[0, :n], accp[1, :n], xw2, dis,
    b.reshape(1, d_out), gamma.reshape(1, d_out), beta.reshape(1, d_out))
  return out
